# baseline (speedup 1.0000x reference)
import functools

import jax
import jax.numpy as jnp
from jax import lax
from jax.experimental import pallas as pl
from jax.experimental.pallas import tpu as pltpu
from jax.experimental.pallas import tpu_sc as plsc

B, C, H, W = 4096, 64, 4, 4
D = C * H * W
MAX_ID = 60000
G = MAX_ID // 32
NC, NS, L = 2, 16, 16
NW = NC * NS
BPW = B // NW
CH = 32
NCHUNK = BPW // CH

_mesh = plsc.VectorSubcoreMesh(core_axis_name="c", subcore_axis_name="s")


@functools.partial(
    pl.kernel,
    mesh=_mesh,
    compiler_params=pltpu.CompilerParams(needs_layout_passes=False),
    out_type=jax.ShapeDtypeStruct((B, D), jnp.float32),
    scratch_types=[
        pltpu.VMEM((BPW,), jnp.int32),
        pltpu.VMEM((BPW,), jnp.int32),
        pltpu.VMEM((CH, D), jnp.int32),
        pltpu.VMEM((CH, D), jnp.float32),
        pltpu.SemaphoreType.DMA,
        pltpu.SemaphoreType.DMA,
    ],
)
def _tied_dropout(x_hbm, idx_hbm, packed_hbm, out_hbm,
                  idx_v, g_v, p_v, x_v, psem, xsem):
    wid = lax.axis_index("s") * NC + lax.axis_index("c")
    base = wid * BPW
    pltpu.sync_copy(idx_hbm.at[pl.ds(base, BPW)], idx_v)

    def gv_body(j, _):
        g_v[pl.ds(j * L, L)] = idx_v[pl.ds(j * L, L)] >> 5
        return 0

    lax.fori_loop(0, BPW // L, gv_body, 0)

    for k in range(NCHUNK):
        row0 = base + k * CH
        pc = pltpu.async_copy(
            packed_hbm.at[g_v.at[pl.ds(k * CH, CH)]], p_v, psem)
        xc = pltpu.async_copy(x_hbm.at[pl.ds(row0, CH)], x_v, xsem)
        pc.wait()
        xc.wait()

        def row_body(r, _):
            jsplat = plsc.load_gather(
                idx_v, [jnp.full((L,), k * CH, jnp.int32) + r])
            jv = jsplat & 31

            def col_body(c, _):
                c0 = c * L
                pw = p_v[r, pl.ds(c0, L)]
                b = ((pw >> jv) & 1).astype(jnp.float32)
                x_v[r, pl.ds(c0, L)] = x_v[r, pl.ds(c0, L)] * b
                return 0

            lax.fori_loop(0, D // L, col_body, 0)
            return 0

        lax.fori_loop(0, CH, row_body, 0)
        pltpu.sync_copy(x_v, out_hbm.at[pl.ds(row0, CH)])


def kernel(X, idx, mask_tensor):
    bits = (mask_tensor != 0).astype(jnp.int32)
    shifts = jnp.arange(32, dtype=jnp.int32).reshape(1, 32, 1, 1, 1)
    packed = jnp.sum(bits.reshape(G, 32, C, H, W) << shifts, axis=1,
                     dtype=jnp.int32)
    x2 = X.reshape(B, D)
    out = _tied_dropout(x2, idx, packed.reshape(G, D))
    return out.reshape(B, C, H, W)
